# baseline (speedup 1.0000x reference)
import functools

import jax
import jax.numpy as jnp
from jax import lax
from jax.experimental import pallas as pl
from jax.experimental.pallas import tpu as pltpu
from jax.experimental.pallas import tpu_sc as plsc


def _make_sc_kernel(B, K, M, DEG, rows_per_w, chunk):
    n_chunks = rows_per_w // chunk
    mesh = plsc.VectorSubcoreMesh(core_axis_name="c", subcore_axis_name="s")

    @functools.partial(
        pl.kernel,
        mesh=mesh,
        out_type=jax.ShapeDtypeStruct((B, M), jnp.int32),
        compiler_params=pltpu.CompilerParams(
            needs_layout_passes=False, use_tc_tiling_on_sc=False
        ),
        scratch_types=[
            pltpu.VMEM((DEG, M), jnp.int32),
            pltpu.VMEM((DEG, M), jnp.int32),
            pltpu.VMEM((chunk, K), jnp.int32),
            pltpu.VMEM((chunk, M), jnp.int32),
        ],
    )
    def k(b_hbm, ps_hbm, ms_hbm, out_hbm, ps_v, ms_v, in_v, out_v):
        nc = 2
        wid = lax.axis_index("s") * nc + lax.axis_index("c")
        base = wid * rows_per_w
        pltpu.sync_copy(ps_hbm, ps_v)
        pltpu.sync_copy(ms_hbm, ms_v)
        idx = [ps_v[j] for j in range(DEG)]
        msk = [ms_v[j] for j in range(DEG)]

        def chunk_body(g, carry):
            row0 = base + g * chunk
            pltpu.sync_copy(b_hbm.at[pl.ds(row0, chunk)], in_v)

            def row_body(r, c2):
                rv = jnp.full((16,), r, dtype=jnp.int32)
                acc = plsc.load_gather(in_v, [rv, idx[0]]) * msk[0]
                for j in range(1, DEG):
                    acc = acc + plsc.load_gather(in_v, [rv, idx[j]]) * msk[j]
                out_v[r] = acc & 1
                return c2

            lax.fori_loop(0, chunk, row_body, 0)
            pltpu.sync_copy(out_v, out_hbm.at[pl.ds(row0, chunk)])
            return carry

        lax.fori_loop(0, n_chunks, chunk_body, 0)

    return k


def kernel(b_info, Ps, Ms):
    B, K = b_info.shape
    M, DEG = Ps.shape
    n_workers = 32
    rows_per_w = B // n_workers
    chunk = 512
    k = _make_sc_kernel(B, K, M, DEG, rows_per_w, chunk)
    return k(
        b_info,
        Ps.T.astype(jnp.int32),
        Ms.T.astype(jnp.int32),
    )


# speedup vs baseline: 1.0555x; 1.0555x over previous
import functools

import jax
import jax.numpy as jnp
from jax import lax
from jax.experimental import pallas as pl
from jax.experimental.pallas import tpu as pltpu
from jax.experimental.pallas import tpu_sc as plsc


def _make_sc_kernel(B, K, M, DEG, rows_per_w, chunk, unroll):
    n_chunks = rows_per_w // chunk
    assert n_chunks % 2 == 0 and chunk % unroll == 0
    mesh = plsc.VectorSubcoreMesh(core_axis_name="c", subcore_axis_name="s")

    @functools.partial(
        pl.kernel,
        mesh=mesh,
        out_type=jax.ShapeDtypeStruct((B, M), jnp.int32),
        compiler_params=pltpu.CompilerParams(
            needs_layout_passes=False, use_tc_tiling_on_sc=False
        ),
        scratch_types=[
            pltpu.VMEM((DEG, M), jnp.int32),
            pltpu.VMEM((DEG, M), jnp.int32),
            pltpu.VMEM((chunk * K,), jnp.int32),
            pltpu.VMEM((chunk * K,), jnp.int32),
            pltpu.VMEM((chunk, M), jnp.int32),
            pltpu.VMEM((chunk, M), jnp.int32),
            pltpu.SemaphoreType.DMA,
            pltpu.SemaphoreType.DMA,
            pltpu.SemaphoreType.DMA,
            pltpu.SemaphoreType.DMA,
        ],
    )
    def k(b_hbm, ps_hbm, ms_hbm, out_hbm, ps_v, ms_v,
          in0, in1, o0, o1, si0, si1, so0, so1):
        nc = 2
        wid = lax.axis_index("s") * nc + lax.axis_index("c")
        base = wid * rows_per_w
        pltpu.sync_copy(ps_hbm, ps_v)
        pltpu.sync_copy(ms_hbm, ms_v)
        idx = [ps_v[j] for j in range(DEG)]
        msk = [ms_v[j] for j in range(DEG)]
        in_bufs = (in0, in1)
        out_bufs = (o0, o1)
        in_sems = (si0, si1)
        out_sems = (so0, so1)

        def in_copy(g, b):
            return pltpu.make_async_copy(
                b_hbm.at[pl.ds((base + g * chunk) * K, chunk * K)],
                in_bufs[b], in_sems[b])

        def out_copy(g, b):
            return pltpu.make_async_copy(
                out_bufs[b], out_hbm.at[pl.ds(base + g * chunk, chunk)],
                out_sems[b])

        in_copy(0, 0).start()

        def pair_body(p, carry):
            g0 = p * 2
            for b in range(2):
                g = g0 + b
                nxt = g + 1

                @pl.when(nxt < n_chunks)
                def _():
                    in_copy(nxt, 1 - b).start()

                in_copy(g, b).wait()

                @pl.when(g >= 2)
                def _():
                    out_copy(g - 2, b).wait()

                in_v = in_bufs[b]
                out_v = out_bufs[b]

                def row_body(i, c2):
                    r0 = i * unroll
                    for u in range(unroll):
                        r = r0 + u
                        rb = r * K
                        acc = plsc.load_gather(in_v, [idx[0] + rb]) * msk[0]
                        for j in range(1, DEG):
                            acc = acc + plsc.load_gather(
                                in_v, [idx[j] + rb]) * msk[j]
                        out_v[r] = acc & 1
                    return c2

                lax.fori_loop(0, chunk // unroll, row_body, 0)

                out_copy(g, b).start()
            return carry

        lax.fori_loop(0, n_chunks // 2, pair_body, 0)
        out_copy(n_chunks - 2, 0).wait()
        out_copy(n_chunks - 1, 1).wait()

    return k


def kernel(b_info, Ps, Ms):
    B, K = b_info.shape
    M, DEG = Ps.shape
    n_workers = 32
    rows_per_w = B // n_workers
    chunk = 1024
    k = _make_sc_kernel(B, K, M, DEG, rows_per_w, chunk, unroll=8)
    return k(
        b_info.reshape(-1),
        Ps.T.astype(jnp.int32),
        Ms.T.astype(jnp.int32),
    )
